# Initial kernel scaffold; baseline (speedup 1.0000x reference)
import jax
import jax.numpy as jnp
from jax.experimental import pallas as pl


def kernel(x_user, x_item, edge_index_rates, edge_index_rev_rates, Wl0r, bl0r, Wr0r, Wl0v, bl0v, Wr0v, g0u, be0u, g0i, be0i, Wl1r, bl1r, Wr1r, Wl1v, bl1v, Wr1v, g1u, be1u, g1i, be1i):
    raise NotImplementedError("write your pallas kernel here")


# speedup vs baseline: 3.0524x; 3.0524x over previous
import functools

import jax
import jax.numpy as jnp
from jax import lax
from jax.experimental import pallas as pl
from jax.experimental.pallas import tpu as pltpu
from jax.experimental.pallas import tpu_sc as plsc

N = 10000
C = 256
E = 160000
EPS = 1e-5

HC = C // 2
K = 128
NCHUNK = E // K
NSUB = 16
NJ = -(-NCHUNK // NSUB)
N_PAD = 10240
ROWS_PT = N_PAD // NSUB
CNT_PAD = N_PAD
BM = 1000


def _zero_vmem_2d(ref, rows, cols):
    def body(i, _):
        r = i // (cols // 16)
        t = i % (cols // 16)
        ref[r, pl.ds(t * 16, 16)] = jnp.zeros((16,), jnp.float32)
        return 0
    lax.fori_loop(0, rows * (cols // 16), body, 0)


def _fill_vmem_1d(ref, n, val):
    def body(i, _):
        ref[pl.ds(i * 16, 16)] = jnp.full((16,), val, jnp.float32)
        return 0
    lax.fori_loop(0, n // 16, body, 0)


def _segsum_body(with_counts, *args):
    if with_counts:
        (x2, src, dst, acc2, cnt, sidx_v, didx_v, rows_v, zc_v, ones_v,
         acc_sh, cnt_sh, sem) = args
    else:
        (x2, src, dst, acc2, sidx_v, didx_v, rows_v, acc_sh, sem) = args

    c = lax.axis_index("c")
    s = lax.axis_index("s")

    _zero_vmem_2d(rows_v, K, HC)
    for k in range(5):
        pltpu.sync_copy(rows_v, acc_sh.at[pl.ds(s * ROWS_PT + k * K, K)])
    if with_counts:
        _fill_vmem_1d(zc_v, 640, 0.0)
        _fill_vmem_1d(ones_v, K, 1.0)

        @pl.when(c == 0)
        def _():
            pltpu.sync_copy(zc_v, cnt_sh.at[pl.ds(s * 640, 640)])

    plsc.subcore_barrier()

    def chunk(j, _):
        cid = s + NSUB * j

        @pl.when(cid < NCHUNK)
        def _():
            base = cid * K
            pltpu.sync_copy(src.at[pl.ds(base, K)], sidx_v)
            pltpu.sync_copy(dst.at[pl.ds(base, K)], didx_v)
            for t in range(K // 16):
                sidx_v[pl.ds(t * 16, 16)] = sidx_v[pl.ds(t * 16, 16)] + c * N
            pltpu.async_copy(x2.at[sidx_v], rows_v, sem).wait()
            pltpu.sync_copy(rows_v, acc_sh.at[didx_v], add=True)
            if with_counts:
                @pl.when(c == 0)
                def _():
                    pltpu.sync_copy(ones_v, cnt_sh.at[didx_v], add=True)
        return 0

    lax.fori_loop(0, NJ, chunk, 0)
    plsc.subcore_barrier()

    for k in range(5):
        r0 = s * ROWS_PT + k * K
        pltpu.sync_copy(acc_sh.at[pl.ds(r0, K)], rows_v)
        pltpu.sync_copy(rows_v, acc2.at[pl.ds(c * N_PAD + r0, K)])
    if with_counts:
        @pl.when(c == 0)
        def _():
            pltpu.sync_copy(cnt_sh.at[pl.ds(s * 640, 640)], zc_v)
            pltpu.sync_copy(zc_v, cnt.at[pl.ds(s * 640, 640)])


def _make_segsum(with_counts):
    mesh = plsc.VectorSubcoreMesh(core_axis_name="c", subcore_axis_name="s")
    out_type = [jax.ShapeDtypeStruct((2 * N_PAD, HC), jnp.float32)]
    scratch = [
        pltpu.VMEM((K,), jnp.int32),
        pltpu.VMEM((K,), jnp.int32),
        pltpu.VMEM((K, HC), jnp.float32),
    ]
    if with_counts:
        out_type.append(jax.ShapeDtypeStruct((CNT_PAD,), jnp.float32))
        scratch += [
            pltpu.VMEM((640,), jnp.float32),
            pltpu.VMEM((K,), jnp.float32),
        ]
    scratch.append(pltpu.VMEM_SHARED((N_PAD, HC), jnp.float32))
    if with_counts:
        scratch.append(pltpu.VMEM_SHARED((CNT_PAD,), jnp.float32))
    scratch.append(pltpu.SemaphoreType.DMA)
    return pl.kernel(
        functools.partial(_segsum_body, with_counts),
        out_type=tuple(out_type) if with_counts else out_type[0],
        mesh=mesh,
        scratch_types=scratch,
    )


_segsum_counts = _make_segsum(True)
_segsum = _make_segsum(False)


def _post_body(stacked_out, acc_ref, cnt_ref, x_ref, wl_ref, wr_ref, b_ref,
               g_ref, be_ref, out_ref):
    acc = jnp.concatenate([acc_ref[0], acc_ref[1]], axis=-1)
    x = jnp.concatenate([x_ref[0], x_ref[1]], axis=-1)
    cnt = jnp.maximum(cnt_ref[...], 1.0)
    mean = acc / cnt
    h = (jnp.dot(mean, wl_ref[...], preferred_element_type=jnp.float32)
         + jnp.dot(x, wr_ref[...], preferred_element_type=jnp.float32)
         + b_ref[...])
    m = jnp.mean(h, axis=-1, keepdims=True)
    v = jnp.mean(jnp.square(h - m), axis=-1, keepdims=True)
    r = jnp.maximum((h - m) * lax.rsqrt(v + EPS) * g_ref[...] + be_ref[...], 0.0)
    if stacked_out:
        out_ref[0] = r[:, :HC]
        out_ref[1] = r[:, HC:]
    else:
        out_ref[...] = r


def _make_post(stacked_out):
    grid = (N // BM,)
    split_spec = pl.BlockSpec((2, BM, HC), lambda i: (0, i, 0))
    full_spec = pl.BlockSpec((C, C), lambda i: (0, 0))
    vec_spec = pl.BlockSpec((1, C), lambda i: (0, 0))
    in_specs = [
        split_spec,
        pl.BlockSpec((BM, 1), lambda i: (i, 0)),
        split_spec,
        full_spec, full_spec,
        vec_spec, vec_spec, vec_spec,
    ]
    if stacked_out:
        out_spec = split_spec
        out_shape = jax.ShapeDtypeStruct((2, N, HC), jnp.float32)
    else:
        out_spec = pl.BlockSpec((BM, C), lambda i: (i, 0))
        out_shape = jax.ShapeDtypeStruct((N, C), jnp.float32)
    return pl.pallas_call(
        functools.partial(_post_body, stacked_out),
        grid=grid,
        in_specs=in_specs,
        out_specs=out_spec,
        out_shape=out_shape,
    )


_post_stacked = _make_post(True)
_post_flat = _make_post(False)


def kernel(x_user, x_item, edge_index_rates, edge_index_rev_rates,
           Wl0r, bl0r, Wr0r, Wl0v, bl0v, Wr0v, g0u, be0u, g0i, be0i,
           Wl1r, bl1r, Wr1r, Wl1v, bl1v, Wr1v, g1u, be1u, g1i, be1i):
    srcR, dstR = edge_index_rates[0], edge_index_rates[1]
    srcV, dstV = edge_index_rev_rates[0], edge_index_rev_rates[1]

    xu2 = x_user.reshape(N, 2, HC).swapaxes(0, 1)
    xi2 = x_item.reshape(N, 2, HC).swapaxes(0, 1)

    aggI0, cntR = _segsum_counts(xu2.reshape(2 * N, HC), srcR, dstR)
    aggU0, cntV = _segsum_counts(xi2.reshape(2 * N, HC), srcV, dstV)
    cntR = cntR[:N].reshape(N, 1)
    cntV = cntV[:N].reshape(N, 1)

    def post(fn, agg, cnt, x2, Wl, bl, Wr, g, be):
        return fn(agg.reshape(2, N_PAD, HC), cnt, x2,
                  Wl.T, Wr.T, bl.reshape(1, C), g.reshape(1, C),
                  be.reshape(1, C))

    it1 = post(_post_stacked, aggI0, cntR, xi2, Wl0r, bl0r, Wr0r, g0i, be0i)
    u1 = post(_post_stacked, aggU0, cntV, xu2, Wl0v, bl0v, Wr0v, g0u, be0u)

    aggI1 = _segsum(u1.reshape(2 * N, HC), srcR, dstR)
    aggU1 = _segsum(it1.reshape(2 * N, HC), srcV, dstV)

    it2 = post(_post_flat, aggI1, cntR, it1, Wl1r, bl1r, Wr1r, g1i, be1i)
    u2 = post(_post_flat, aggU1, cntV, u1, Wl1v, bl1v, Wr1v, g1u, be1u)
    return (u2, it2)
